# Initial kernel scaffold; baseline (speedup 1.0000x reference)
import functools

import jax
import jax.numpy as jnp
from jax.experimental import pallas as pl
from jax.experimental.pallas import tpu as pltpu

NUM_CLASSES = 1000
FEAT_DIM = 2048
SLOTS = 5
TOP_K = 3
B = 4096
QB = 16
NEG = -1e30


def _body(lbl_ref, q_ref, scores_ref, *refs):
    k_refs = refs[:QB]
    v_refs = refs[QB:2 * QB]
    ret_ref, w_ref = refs[2 * QB], refs[2 * QB + 1]

    i = pl.program_id(0)
    q = q_ref[...]
    K = jnp.concatenate([r[...] for r in k_refs], 0)
    V = jnp.concatenate([r[...] for r in v_refs], 0)

    score_rows = [scores_ref[pl.ds(lbl_ref[i * QB + j], 1), :] for j in range(QB)]

    qq = jnp.sum(q * q, axis=-1)
    qn = jnp.maximum(jnp.sqrt(qq), 1e-8)

    combined = []
    scores_cols = []
    for s in range(SLOTS):
        ks = K[:, s, :]
        dot = jnp.sum(ks * q, axis=-1)
        kn = jnp.maximum(jnp.sqrt(jnp.sum(ks * ks, axis=-1)), 1e-8)
        sim = dot / (qn * kn)
        sc = jnp.concatenate([r[:, s] for r in score_rows], 0)
        scores_cols.append(sc)
        combined.append(sim * sc)

    work = list(combined)
    top_scores = []
    slot_weight = [jnp.zeros((QB,), jnp.float32) for _ in range(SLOTS)]
    onehots = []
    for _ in range(TOP_K):
        m = work[0]
        for s in range(1, SLOTS):
            m = jnp.maximum(m, work[s])
        taken = jnp.zeros((QB,), jnp.bool_)
        oh = []
        for s in range(SLOTS):
            is_first = (work[s] == m) & (~taken)
            taken = taken | is_first
            oh.append(is_first)
            work[s] = jnp.where(is_first, NEG, work[s])
        onehots.append(oh)
        top_scores.append(m)

    exps = [jnp.exp((t - top_scores[0]) / 0.1) for t in top_scores]
    denom = exps[0] + exps[1] + exps[2]
    attn = [e / denom for e in exps]

    for k in range(TOP_K):
        for s in range(SLOTS):
            slot_weight[s] = slot_weight[s] + jnp.where(
                onehots[k][s], attn[k], 0.0)

    retrieved = jnp.zeros((QB, FEAT_DIM), jnp.float32)
    for s in range(SLOTS):
        retrieved = retrieved + V[:, s, :] * slot_weight[s][:, None]

    weights = (top_scores[0] + top_scores[1] + top_scores[2]) / 3.0

    ssum = scores_cols[0]
    for s in range(1, SLOTS):
        ssum = ssum + scores_cols[s]
    hit = ssum > 0

    ret_ref[...] = jnp.where(hit[:, None], retrieved, 0.0)
    w_ref[...] = jnp.where(hit, weights, 0.0)[:, None]


def kernel(query, labels, mem_keys, mem_vals, mem_scores):
    labels = labels.astype(jnp.int32)
    grid = B // QB

    def key_spec(j):
        return pl.BlockSpec(
            (1, SLOTS, FEAT_DIM),
            lambda i, lbl, j=j: (lbl[i * QB + j], 0, 0))

    in_specs = (
        [pl.BlockSpec((QB, FEAT_DIM), lambda i, lbl: (i, 0))] +
        [pl.BlockSpec((NUM_CLASSES, SLOTS), lambda i, lbl: (0, 0))] +
        [key_spec(j) for j in range(QB)] +
        [key_spec(j) for j in range(QB)]
    )
    out_specs = [
        pl.BlockSpec((QB, FEAT_DIM), lambda i, lbl: (i, 0)),
        pl.BlockSpec((QB, 1), lambda i, lbl: (i, 0)),
    ]
    grid_spec = pltpu.PrefetchScalarGridSpec(
        num_scalar_prefetch=1,
        grid=(grid,),
        in_specs=in_specs,
        out_specs=out_specs,
    )
    retrieved, weights = pl.pallas_call(
        _body,
        grid_spec=grid_spec,
        out_shape=[
            jax.ShapeDtypeStruct((B, FEAT_DIM), jnp.float32),
            jax.ShapeDtypeStruct((B, 1), jnp.float32),
        ],
    )(labels, query, mem_scores,
      *([mem_keys] * QB), *([mem_vals] * QB))
    return retrieved, weights.reshape(B)


# speedup vs baseline: 1.3037x; 1.3037x over previous
import functools

import jax
import jax.numpy as jnp
from jax.experimental import pallas as pl
from jax.experimental.pallas import tpu as pltpu

NUM_CLASSES = 1000
FEAT_DIM = 2048
SLOTS = 5
TOP_K = 3
B = 4096
QB = 16
R = QB * SLOTS
NEG = -1e30


def _body(lbl_ref, q_ref, scores_ref, *refs):
    k_refs = refs[:QB]
    v_refs = refs[QB:2 * QB]
    ret_ref, w_ref = refs[2 * QB], refs[2 * QB + 1]

    i = pl.program_id(0)
    q = q_ref[...]
    K = jnp.concatenate(
        [r[...].reshape(SLOTS, FEAT_DIM) for r in k_refs], 0)
    V = jnp.concatenate(
        [r[...].reshape(SLOTS, FEAT_DIM) for r in v_refs], 0)

    ones_col = jnp.ones((FEAT_DIM, 1), jnp.float32)

    dots_full = jax.lax.dot_general(
        q, K, (((1,), (1,)), ((), ())),
        preferred_element_type=jnp.float32,
        precision=jax.lax.Precision.HIGHEST)

    qq = jax.lax.dot_general(
        q * q, ones_col, (((1,), (0,)), ((), ())),
        preferred_element_type=jnp.float32,
        precision=jax.lax.Precision.HIGHEST)
    qn = jnp.maximum(jnp.sqrt(qq), 1e-8)

    kk = jax.lax.dot_general(
        K * K, ones_col, (((1,), (0,)), ((), ())),
        preferred_element_type=jnp.float32,
        precision=jax.lax.Precision.HIGHEST)
    kn = jnp.maximum(jnp.sqrt(kk), 1e-8)

    row = jax.lax.broadcasted_iota(jnp.int32, (QB, R), 0)
    col = jax.lax.broadcasted_iota(jnp.int32, (QB, R), 1)
    inv_kn = (1.0 / kn).reshape(1, R)
    sims_full = dots_full * inv_kn / qn

    score_rows = [scores_ref[pl.ds(lbl_ref[i * QB + j], 1), :]
                  for j in range(QB)]
    scores = jnp.concatenate(score_rows, 0)
    scores_tiled = jnp.concatenate([scores] * QB, 1)

    mine = (col // SLOTS) == row
    combined_full = jnp.where(mine, sims_full * scores_tiled, NEG)

    work = combined_full
    top_scores = []
    onehots = []
    for _ in range(TOP_K):
        m = jnp.max(work, axis=1, keepdims=True)
        eq = (work == m) & mine
        idx_of = jnp.min(jnp.where(eq, col, R), axis=1, keepdims=True)
        first = col == idx_of
        top_scores.append(m)
        onehots.append(first)
        work = jnp.where(first, NEG, work)

    exps = [jnp.exp((t - top_scores[0]) / 0.1) for t in top_scores]
    denom = exps[0] + exps[1] + exps[2]
    attn = [e / denom for e in exps]

    W = jnp.zeros((QB, R), jnp.float32)
    for k in range(TOP_K):
        W = W + jnp.where(onehots[k], attn[k], 0.0)

    hit = jnp.sum(scores, axis=1, keepdims=True) > 0
    weights = (top_scores[0] + top_scores[1] + top_scores[2]) / 3.0
    W = jnp.where(hit, W, 0.0)

    retrieved = jax.lax.dot_general(
        W, V, (((1,), (0,)), ((), ())),
        preferred_element_type=jnp.float32,
        precision=jax.lax.Precision.HIGHEST)

    ret_ref[...] = retrieved
    w_ref[...] = jnp.where(hit, weights, 0.0)


def kernel(query, labels, mem_keys, mem_vals, mem_scores):
    labels = labels.astype(jnp.int32)
    grid = B // QB

    def key_spec(j):
        return pl.BlockSpec(
            (1, SLOTS, FEAT_DIM),
            lambda i, lbl, j=j: (lbl[i * QB + j], 0, 0))

    in_specs = (
        [pl.BlockSpec((QB, FEAT_DIM), lambda i, lbl: (i, 0))] +
        [pl.BlockSpec((NUM_CLASSES, SLOTS), lambda i, lbl: (0, 0))] +
        [key_spec(j) for j in range(QB)] +
        [key_spec(j) for j in range(QB)]
    )
    out_specs = [
        pl.BlockSpec((QB, FEAT_DIM), lambda i, lbl: (i, 0)),
        pl.BlockSpec((QB, 1), lambda i, lbl: (i, 0)),
    ]
    grid_spec = pltpu.PrefetchScalarGridSpec(
        num_scalar_prefetch=1,
        grid=(grid,),
        in_specs=in_specs,
        out_specs=out_specs,
    )
    retrieved, weights = pl.pallas_call(
        _body,
        grid_spec=grid_spec,
        out_shape=[
            jax.ShapeDtypeStruct((B, FEAT_DIM), jnp.float32),
            jax.ShapeDtypeStruct((B, 1), jnp.float32),
        ],
    )(labels, query, mem_scores,
      *([mem_keys] * QB), *([mem_vals] * QB))
    return retrieved, weights.reshape(B)
